# baseline (speedup 1.0000x reference)
import functools

import jax
import jax.numpy as jnp
from jax import lax
from jax.experimental import pallas as pl
from jax.experimental.pallas import tpu as pltpu
from jax.experimental.pallas import tpu_sc as plsc

NC, NS, LANES = 2, 16, 16
NW = NC * NS

B, L, DIM = 4096, 200, 64
N = B * L
NPW = N // NW
CH = 512
NCHUNK = NPW // CH
GSEG = 128
XCH = 6400
NXCH = NPW // XCH

_mesh = plsc.VectorSubcoreMesh(
    core_axis_name="c", subcore_axis_name="s", num_cores=NC, num_subcores=NS)


@functools.partial(
    pl.kernel,
    out_type=jax.ShapeDtypeStruct((N, DIM), jnp.float32),
    mesh=_mesh,
    scratch_types=[
        pltpu.VMEM((XCH,), jnp.int32),
        pltpu.VMEM((XCH,), jnp.int32),
        pltpu.VMEM((NPW,), jnp.int32),
        pltpu.VMEM((CH, DIM), jnp.float32),
        pltpu.VMEM((CH, DIM), jnp.float32),
        pltpu.SemaphoreType.DMA,
        pltpu.SemaphoreType.DMA,
        pltpu.SemaphoreType.DMA,
        pltpu.SemaphoreType.DMA,
    ],
    compiler_params=pltpu.CompilerParams(use_tc_tiling_on_sc=False),
)
def _sc_embed(brick_hbm, rot_hbm, emb_hbm, out_hbm, bb, rb, idxb,
              rows0, rows1, gs0, gs1, ss0, ss1):
    wid = lax.axis_index("s") * NC + lax.axis_index("c")
    base = wid * NPW

    def xloop(xc, carry):
        pltpu.sync_copy(brick_hbm.at[pl.ds(base + xc * XCH, XCH)], bb)
        pltpu.sync_copy(rot_hbm.at[pl.ds(base + xc * XCH, XCH)], rb)

        def jloop(j, c2):
            brick = bb[pl.ds(j * LANES, LANES)]
            rot = rb[pl.ds(j * LANES, LANES)]
            idx = (1 + brick) * (1 + ((rot * 3) >> 8))
            idxb[pl.ds(xc * XCH + j * LANES, LANES)] = idx
            return c2

        return lax.fori_loop(0, XCH // LANES, jloop, carry)

    lax.fori_loop(0, NXCH, xloop, 0)

    def chunk_body(c, buf, g_sem, s_sem):
        row0 = base + c * CH
        @pl.when(c >= 2)
        def _():
            pltpu.make_async_copy(
                buf, out_hbm.at[pl.ds(row0, CH)], s_sem).wait()

        descs = []
        for k in range(CH // GSEG):
            idxs = idxb.at[pl.ds(c * CH + k * GSEG, GSEG)]
            descs.append(pltpu.async_copy(
                emb_hbm.at[idxs], buf.at[pl.ds(k * GSEG, GSEG)], g_sem))
        for d in descs:
            d.wait()
        pltpu.async_copy(buf, out_hbm.at[pl.ds(row0, CH)], s_sem)

    def pair(p, carry):
        chunk_body(2 * p, rows0, gs0, ss0)
        chunk_body(2 * p + 1, rows1, gs1, ss1)
        return carry

    lax.fori_loop(0, NCHUNK // 2, pair, 0)
    pltpu.make_async_copy(rows0, out_hbm.at[pl.ds(base, CH)], ss0).wait()
    pltpu.make_async_copy(rows1, out_hbm.at[pl.ds(base, CH)], ss1).wait()


def kernel(x, emb):
    xi = x.astype(jnp.int32)
    brick = xi[..., 0].reshape(N)
    rot = xi[..., 1].reshape(N)
    out = _sc_embed(brick, rot, emb)
    return out.reshape(B, L, DIM)


# speedup vs baseline: 14.1938x; 14.1938x over previous
import functools

import jax
import jax.numpy as jnp
from jax import lax
from jax.experimental import pallas as pl
from jax.experimental.pallas import tpu as pltpu
from jax.experimental.pallas import tpu_sc as plsc

NC, NS, LANES = 2, 16, 16
NW = NC * NS

B, L, DIM = 4096, 200, 64
N = B * L
NPW = N // NW
CH = 512
NCHUNK = NPW // CH
GSEG = 128
XCH = 6400
NXCH = NPW // XCH

_mesh = plsc.VectorSubcoreMesh(
    core_axis_name="c", subcore_axis_name="s", num_cores=NC, num_subcores=NS)


@functools.partial(
    pl.kernel,
    out_type=jax.ShapeDtypeStruct((N, DIM), jnp.float32),
    mesh=_mesh,
    scratch_types=[
        pltpu.VMEM((XCH,), jnp.int32),
        pltpu.VMEM((XCH,), jnp.int32),
        pltpu.VMEM((NPW,), jnp.int32),
        pltpu.VMEM((CH, DIM), jnp.float32),
        pltpu.VMEM((CH, DIM), jnp.float32),
        pltpu.VMEM_SHARED((5, DIM), jnp.float32),
        pltpu.SemaphoreType.DMA,
        pltpu.SemaphoreType.DMA,
        pltpu.SemaphoreType.DMA,
        pltpu.SemaphoreType.DMA,
    ],
    compiler_params=pltpu.CompilerParams(use_tc_tiling_on_sc=False),
)
def _sc_embed(brick_hbm, rot_hbm, emb_hbm, out_hbm, bb, rb, idxb,
              rows0, rows1, emb_sh, gs0, gs1, ss0, ss1):
    wid = lax.axis_index("s") * NC + lax.axis_index("c")
    base = wid * NPW

    @pl.when(lax.axis_index("s") == 0)
    def _():
        pltpu.sync_copy(emb_hbm, emb_sh)

    plsc.subcore_barrier()

    def xloop(xc, carry):
        pltpu.sync_copy(brick_hbm.at[pl.ds(base + xc * XCH, XCH)], bb)
        pltpu.sync_copy(rot_hbm.at[pl.ds(base + xc * XCH, XCH)], rb)

        def jloop(j, c2):
            brick = bb[pl.ds(j * LANES, LANES)]
            rot = rb[pl.ds(j * LANES, LANES)]
            idx = (1 + brick) * (1 + ((rot * 3) >> 8))
            idxb[pl.ds(xc * XCH + j * LANES, LANES)] = idx
            return c2

        return lax.fori_loop(0, XCH // LANES, jloop, carry)

    lax.fori_loop(0, NXCH, xloop, 0)

    def chunk_body(c, buf, g_sem, s_sem):
        row0 = base + c * CH
        @pl.when(c >= 2)
        def _():
            pltpu.make_async_copy(
                buf, out_hbm.at[pl.ds(row0, CH)], s_sem).wait()

        descs = []
        for k in range(CH // GSEG):
            idxs = idxb.at[pl.ds(c * CH + k * GSEG, GSEG)]
            descs.append(pltpu.async_copy(
                emb_sh.at[idxs], buf.at[pl.ds(k * GSEG, GSEG)], g_sem))
        for d in descs:
            d.wait()
        pltpu.async_copy(buf, out_hbm.at[pl.ds(row0, CH)], s_sem)

    def pair(p, carry):
        chunk_body(2 * p, rows0, gs0, ss0)
        chunk_body(2 * p + 1, rows1, gs1, ss1)
        return carry

    lax.fori_loop(0, NCHUNK // 2, pair, 0)
    pltpu.make_async_copy(rows0, out_hbm.at[pl.ds(base, CH)], ss0).wait()
    pltpu.make_async_copy(rows1, out_hbm.at[pl.ds(base, CH)], ss1).wait()


def kernel(x, emb):
    xi = x.astype(jnp.int32)
    brick = xi[..., 0].reshape(N)
    rot = xi[..., 1].reshape(N)
    out = _sc_embed(brick, rot, emb)
    return out.reshape(B, L, DIM)
